# Initial kernel scaffold; baseline (speedup 1.0000x reference)
import jax
import jax.numpy as jnp
from jax.experimental import pallas as pl


def kernel(attn_bias, spatial_pos, x, edge_input, attn_edge_type, spatial_W):
    raise NotImplementedError("write your pallas kernel here")


# speedup vs baseline: 4.9299x; 4.9299x over previous
import functools

import jax
import jax.numpy as jnp
from jax import lax
from jax.experimental import pallas as pl
from jax.experimental.pallas import tpu as pltpu
from jax.experimental.pallas import tpu_sc as plsc

NH = 32
S = 513
NB = 16
ROWS = NB * S
VECS = (S + 15) // 16
PADW = VECS * 16
WSTRIDE = 520
NW = 32


def _sc_body(ab_hbm, sp_hbm, wt_hbm, out_hbm, wcols, ab_v, idx_v, outbuf):
    wid = lax.axis_index("s") * 2 + lax.axis_index("c")
    pltpu.sync_copy(wt_hbm, wcols)
    idx_v[pl.ds(512, 16)] = jnp.zeros((16,), jnp.int32)

    def row_body(t, carry):
        r = wid + t * NW

        @pl.when(r < ROWS)
        def _():
            pltpu.sync_copy(ab_hbm.at[r], ab_v.at[pl.ds(0, S)])
            pltpu.sync_copy(sp_hbm.at[r], idx_v.at[pl.ds(0, S)])

            def vec_body(jv, c):
                off = jv * 16
                idx = idx_v[pl.ds(off, 16)]
                ab = ab_v[pl.ds(off, 16)]
                ab2 = ab + ab
                for h in range(NH):
                    g = plsc.load_gather(wcols, [idx + (h * WSTRIDE)])
                    outbuf[pl.ds(h * PADW + off, 16)] = ab2 + g
                return c

            lax.fori_loop(0, VECS, vec_body, 0)
            n = r // S
            i = r - n * S
            obase = n * (NH * S) + i
            for h in range(NH):
                pltpu.sync_copy(outbuf.at[pl.ds(h * PADW, S)],
                                out_hbm.at[obase + h * S])

        return carry

    ntasks = (ROWS + NW - 1) // NW
    lax.fori_loop(0, ntasks, row_body, 0)


@jax.jit
def _sc_call(ab2, sp2, wt):
    mesh = plsc.VectorSubcoreMesh(core_axis_name="c", subcore_axis_name="s")
    f = pl.kernel(
        _sc_body,
        out_type=jax.ShapeDtypeStruct((NB * NH * S, S), jnp.float32),
        mesh=mesh,
        compiler_params=pltpu.CompilerParams(needs_layout_passes=False,
                                             use_tc_tiling_on_sc=False),
        scratch_types=[
            pltpu.VMEM((NH * WSTRIDE,), jnp.float32),
            pltpu.VMEM((PADW,), jnp.float32),
            pltpu.VMEM((PADW,), jnp.int32),
            pltpu.VMEM((NH * PADW,), jnp.float32),
        ],
    )
    return f(ab2, sp2, wt)


def kernel(attn_bias, spatial_pos, x, edge_input, attn_edge_type, spatial_W):
    del x, edge_input, attn_edge_type
    W0 = spatial_W.at[0].set(0.0)
    wt = jnp.pad(W0.T, ((0, 0), (0, WSTRIDE - S))).reshape(-1)
    sp_pad = jnp.pad(spatial_pos, ((0, 0), (1, 0), (1, 0)))
    ab2 = attn_bias.reshape(NB * S, S)
    sp2 = sp_pad.reshape(NB * S, S)
    out2 = _sc_call(ab2, sp2, wt)
    return out2.reshape(NB, NH, S, S)


# speedup vs baseline: 6.2024x; 1.2581x over previous
import functools

import jax
import jax.numpy as jnp
from jax import lax
from jax.experimental import pallas as pl
from jax.experimental.pallas import tpu as pltpu
from jax.experimental.pallas import tpu_sc as plsc

NH = 32
S = 513
NB = 16
ROWS = NB * S
VECS = (S + 15) // 16
PADW = VECS * 16
WSTRIDE = 520
NW = 32
OBUF = NH * PADW


def _sc_body(ab_hbm, sp_hbm, wt_hbm, out_hbm, wcols, ab_v, idx_v, outbuf,
             insem0, insem1, outsem0, outsem1):
    wid = lax.axis_index("s") * 2 + lax.axis_index("c")
    pltpu.sync_copy(wt_hbm, wcols)
    idx_v[pl.ds(512, 16)] = jnp.zeros((16,), jnp.int32)
    idx_v[pl.ds(PADW + 512, 16)] = jnp.zeros((16,), jnp.int32)

    ntasks = (ROWS + NW - 1) // NW

    def in_copies(t, p):
        r = wid + t * NW
        return (
            pltpu.make_async_copy(ab_hbm.at[r],
                                  ab_v.at[pl.ds(p * PADW, S)],
                                  insem0 if p == 0 else insem1),
            pltpu.make_async_copy(sp_hbm.at[r],
                                  idx_v.at[pl.ds(p * PADW, S)],
                                  insem0 if p == 0 else insem1),
        )

    def out_copy(t, p, h):
        r = wid + t * NW
        n = r // S
        obase = n * (NH * S) + (r - n * S)
        return pltpu.make_async_copy(
            outbuf.at[pl.ds(p * OBUF + h * PADW, S)],
            out_hbm.at[obase + h * S],
            outsem0 if p == 0 else outsem1)

    def start_in(t, p):
        for c in in_copies(t, p):
            c.start()

    start_in(0, 0)

    def task_body(t, p):
        r = wid + t * NW
        @pl.when(r + NW < ROWS)
        def _():
            start_in(t + 1, 1 - p)
        for c in in_copies(t, p):
            c.wait()
        @pl.when(t >= 2)
        def _():
            for h in range(NH):
                out_copy(t - 2, p, h).wait()

        def vec_body(jv, c):
            off = jv * 16
            idx = idx_v[pl.ds(p * PADW + off, 16)]
            ab = ab_v[pl.ds(p * PADW + off, 16)]
            ab2 = ab + ab
            for h in range(NH):
                g = plsc.load_gather(wcols, [idx + (h * WSTRIDE)])
                outbuf[pl.ds(p * OBUF + h * PADW + off, 16)] = ab2 + g
            return c

        lax.fori_loop(0, VECS, vec_body, 0)
        for h in range(NH):
            out_copy(t, p, h).start()

    def even_odd(tt, carry):
        te = 2 * tt

        @pl.when(wid + te * NW < ROWS)
        def _():
            task_body(te, 0)

        @pl.when(te + 1 < ntasks)
        def _():
            task_body(te + 1, 1)

        return carry

    lax.fori_loop(0, (ntasks + 1) // 2, even_odd, 0)

    for h in range(NH):
        out_copy(ntasks - 2, 1, h).wait()

    ragged = wid + (ntasks - 1) * NW < ROWS

    @pl.when(ragged)
    def _():
        for h in range(NH):
            out_copy(ntasks - 1, 0, h).wait()

    @pl.when(jnp.logical_not(ragged))
    def _():
        for h in range(NH):
            out_copy(ntasks - 3, 0, h).wait()


@jax.jit
def _sc_call(ab2, sp2, wt):
    mesh = plsc.VectorSubcoreMesh(core_axis_name="c", subcore_axis_name="s")
    f = pl.kernel(
        _sc_body,
        out_type=jax.ShapeDtypeStruct((NB * NH * S, S), jnp.float32),
        mesh=mesh,
        compiler_params=pltpu.CompilerParams(needs_layout_passes=False,
                                             use_tc_tiling_on_sc=False),
        scratch_types=[
            pltpu.VMEM((NH * WSTRIDE,), jnp.float32),
            pltpu.VMEM((2 * PADW,), jnp.float32),
            pltpu.VMEM((2 * PADW,), jnp.int32),
            pltpu.VMEM((2 * OBUF,), jnp.float32),
            pltpu.SemaphoreType.DMA,
            pltpu.SemaphoreType.DMA,
            pltpu.SemaphoreType.DMA,
            pltpu.SemaphoreType.DMA,
        ],
    )
    return f(ab2, sp2, wt)


def kernel(attn_bias, spatial_pos, x, edge_input, attn_edge_type, spatial_W):
    del x, edge_input, attn_edge_type
    W0 = spatial_W.at[0].set(0.0)
    wt = jnp.pad(W0.T, ((0, 0), (0, WSTRIDE - S))).reshape(-1)
    sp_pad = jnp.pad(spatial_pos, ((0, 0), (1, 0), (1, 0)))
    ab2 = attn_bias.reshape(NB * S, S)
    sp2 = sp_pad.reshape(NB * S, S)
    out2 = _sc_call(ab2, sp2, wt)
    return out2.reshape(NB, NH, S, S)
